# baseline (speedup 1.0000x reference)
import functools

import jax
import jax.numpy as jnp
from jax import lax
from jax.experimental import pallas as pl
from jax.experimental.pallas import tpu as pltpu
from jax.experimental.pallas import tpu_sc as plsc

_B, _S, _D = 4, 4096, 1024
_NC, _NS, _L = 2, 16, 16
_NW = _NC * _NS
_ROWS_PER_W = _S // _NW
_R = 4
_CHUNK = _R * _D
_N_CHUNKS = _ROWS_PER_W // _R


def _sc_body(x_hbm, pe_hbm, out_hbm, *refs):
    pe_bufs = refs[0:2]
    x_bufs = (refs[2:6], refs[6:10])
    o_bufs = (refs[10:14], refs[14:18])
    pe_sems = refs[18:20]
    x_sems = (refs[20:24], refs[24:28])
    o_sems = (refs[28:32], refs[32:36])

    wid = lax.axis_index("s") * _NC + lax.axis_index("c")
    base0 = wid * (_ROWS_PER_W * _D)

    def start_pe(c):
        return pltpu.async_copy(
            pe_hbm.at[pl.ds(base0 + c * _CHUNK, _CHUNK)],
            pe_bufs[c % 2], pe_sems[c % 2])

    def start_x(c, b):
        src = x_hbm.at[pl.ds(b * (_S * _D) + base0 + c * _CHUNK, _CHUNK)]
        return pltpu.async_copy(src, x_bufs[c % 2][b], x_sems[c % 2][b])

    def start_out(c, b):
        dst = out_hbm.at[pl.ds(b * (_S * _D) + base0 + c * _CHUNK, _CHUNK)]
        return pltpu.async_copy(o_bufs[c % 2][b], dst, o_sems[c % 2][b])

    pe_dma = [None] * (_N_CHUNKS + 1)
    x_dma = [[None] * _B for _ in range(_N_CHUNKS + 1)]
    o_dma = [[None] * _B for _ in range(_N_CHUNKS + 1)]

    pe_dma[0] = start_pe(0)
    for b in range(_B):
        x_dma[0][b] = start_x(0, b)

    for c in range(_N_CHUNKS):
        if c + 1 < _N_CHUNKS:
            pe_dma[c + 1] = start_pe(c + 1)
            for b in range(_B):
                x_dma[c + 1][b] = start_x(c + 1, b)

        pe_dma[c].wait()
        for b in range(_B):
            x_dma[c][b].wait()
        if c >= 2:
            for b in range(_B):
                o_dma[c - 2][b].wait()

        pe_v = pe_bufs[c % 2]
        x_v = x_bufs[c % 2]
        o_v = o_bufs[c % 2]

        @plsc.parallel_loop(0, _CHUNK, step=_L, unroll=4)
        def add_body(i, pe_v=pe_v, x_v=x_v, o_v=o_v):
            sl = pl.ds(i, _L)
            pe_r = pe_v[sl]
            for b in range(_B):
                o_v[b][sl] = x_v[b][sl] + pe_r

        for b in range(_B):
            o_dma[c][b] = start_out(c, b)

    for c in (_N_CHUNKS - 2, _N_CHUNKS - 1):
        for b in range(_B):
            o_dma[c][b].wait()


@jax.jit
def _sc_call(x_flat, pe_flat):
    mesh = plsc.VectorSubcoreMesh(core_axis_name="c", subcore_axis_name="s")
    scratch = (
        [pltpu.VMEM((_CHUNK,), jnp.float32) for _ in range(18)]
        + [pltpu.SemaphoreType.DMA for _ in range(18)]
    )
    k = functools.partial(
        pl.kernel,
        mesh=mesh,
        out_type=jax.ShapeDtypeStruct((_B * _S * _D,), jnp.float32),
        scratch_types=scratch,
    )(_sc_body)
    return k(x_flat, pe_flat)


def kernel(x, pos_emb):
    B, S, D = x.shape
    pe = pos_emb[:S]
    out_flat = _sc_call(x.reshape(-1), pe.reshape(-1))
    return out_flat.reshape(B, S, D)


# speedup vs baseline: 1.0355x; 1.0355x over previous
import functools

import jax
import jax.numpy as jnp
from jax import lax
from jax.experimental import pallas as pl
from jax.experimental.pallas import tpu as pltpu
from jax.experimental.pallas import tpu_sc as plsc

_B, _S, _D = 4, 4096, 1024
_NC, _NS, _L = 2, 16, 16
_NW = _NC * _NS
_ROWS_PER_W = _S // _NW
_R = 8
_CHUNK = _R * _D
_N_CHUNKS = _ROWS_PER_W // _R
_UNITS = _N_CHUNKS * _B
_DEPTH = 4


def _sc_body(x_hbm, pe_hbm, out_hbm, *refs):
    pe_bufs = refs[0:2]
    x_bufs = refs[2:2 + _DEPTH]
    o_bufs = refs[6:6 + _DEPTH]
    pe_sems = refs[10:12]
    x_sems = refs[12:12 + _DEPTH]
    o_sems = refs[16:16 + _DEPTH]

    wid = lax.axis_index("s") * _NC + lax.axis_index("c")
    base0 = wid * (_ROWS_PER_W * _D)

    def start_pe(c):
        return pltpu.async_copy(
            pe_hbm.at[pl.ds(base0 + c * _CHUNK, _CHUNK)],
            pe_bufs[c % 2], pe_sems[c % 2])

    def start_x(u):
        c, b = divmod(u, _B)
        src = x_hbm.at[pl.ds(b * (_S * _D) + base0 + c * _CHUNK, _CHUNK)]
        return pltpu.async_copy(src, x_bufs[u % _DEPTH], x_sems[u % _DEPTH])

    def start_out(u):
        c, b = divmod(u, _B)
        dst = out_hbm.at[pl.ds(b * (_S * _D) + base0 + c * _CHUNK, _CHUNK)]
        return pltpu.async_copy(o_bufs[u % _DEPTH], dst, o_sems[u % _DEPTH])

    pe_dma = [None] * (_N_CHUNKS + 1)
    x_dma = [None] * (_UNITS + 1)
    o_dma = [None] * (_UNITS + 1)

    pe_dma[0] = start_pe(0)
    for u in range(_DEPTH):
        x_dma[u] = start_x(u)

    for u in range(_UNITS):
        c, b = divmod(u, _B)
        if b == 0 and c + 1 < _N_CHUNKS:
            pe_dma[c + 1] = start_pe(c + 1)

        x_dma[u].wait()
        if b == 0:
            pe_dma[c].wait()
        if u >= _DEPTH:
            o_dma[u - _DEPTH].wait()

        pe_v = pe_bufs[c % 2]
        x_v = x_bufs[u % _DEPTH]
        o_v = o_bufs[u % _DEPTH]

        @plsc.parallel_loop(0, _CHUNK, step=_L, unroll=8)
        def add_body(i, pe_v=pe_v, x_v=x_v, o_v=o_v):
            sl = pl.ds(i, _L)
            o_v[sl] = x_v[sl] + pe_v[sl]

        o_dma[u] = start_out(u)
        if u + _DEPTH < _UNITS:
            x_dma[u + _DEPTH] = start_x(u + _DEPTH)

    for u in range(_UNITS - _DEPTH, _UNITS):
        o_dma[u].wait()


@jax.jit
def _sc_call(x_flat, pe_flat):
    mesh = plsc.VectorSubcoreMesh(core_axis_name="c", subcore_axis_name="s")
    scratch = (
        [pltpu.VMEM((_CHUNK,), jnp.float32) for _ in range(2 + 2 * _DEPTH)]
        + [pltpu.SemaphoreType.DMA for _ in range(2 + 2 * _DEPTH)]
    )
    k = functools.partial(
        pl.kernel,
        mesh=mesh,
        out_type=jax.ShapeDtypeStruct((_B * _S * _D,), jnp.float32),
        scratch_types=scratch,
    )(_sc_body)
    return k(x_flat, pe_flat)


def kernel(x, pos_emb):
    B, S, D = x.shape
    pe = pos_emb[:S]
    out_flat = _sc_call(x.reshape(-1), pe.reshape(-1))
    return out_flat.reshape(B, S, D)


# speedup vs baseline: 4.5736x; 4.4167x over previous
import jax
import jax.numpy as jnp
from jax.experimental import pallas as pl


def _pe_add_body(x_ref, pe_ref, o_ref):
    o_ref[...] = x_ref[...] + pe_ref[...]


def kernel(x, pos_emb):
    B, S, D = x.shape
    SB = 256
    pe = pos_emb[:S]
    return pl.pallas_call(
        _pe_add_body,
        grid=(S // SB,),
        in_specs=[
            pl.BlockSpec((B, SB, D), lambda s: (0, s, 0)),
            pl.BlockSpec((SB, D), lambda s: (s, 0)),
        ],
        out_specs=pl.BlockSpec((B, SB, D), lambda s: (0, s, 0)),
        out_shape=jax.ShapeDtypeStruct((B, S, D), x.dtype),
    )(x, pe)
